# baseline (speedup 1.0000x reference)
import functools

import jax
import jax.numpy as jnp
from jax import lax
from jax.experimental import pallas as pl
from jax.experimental.pallas import tpu as pltpu
from jax.experimental.pallas import tpu_sc as plsc

_NC = 2
_NS = 16
_NW = _NC * _NS
_L = 16
_C = 8
_NSLOT = 3


@functools.lru_cache(maxsize=None)
def _build_embed(Bn, S, V, D):
    assert S % (_NW * _C) == 0 and D % _L == 0
    SW = S // _NW
    NCH = SW // _C
    VECS = D // _L
    BS = Bn * S
    R = Bn * _C
    assert NCH % 3 == 2 and NCH >= 8

    mesh = plsc.VectorSubcoreMesh(core_axis_name="c", subcore_axis_name="s")
    scratch = (
        [pltpu.VMEM((SW * Bn,), jnp.int32)]
        + [pltpu.VMEM((_C, D), jnp.float32) for _ in range(_NSLOT)]
        + [pltpu.VMEM((R, D), jnp.float32) for _ in range(_NSLOT)]
        + [pltpu.SemaphoreType.DMA for _ in range(3 * _NSLOT)]
    )

    @functools.partial(
        pl.kernel,
        mesh=mesh,
        out_type=jax.ShapeDtypeStruct((BS, D), jnp.float32),
        scratch_types=scratch,
    )
    def embed(idx_hbm, tok_hbm, pos_hbm, out_hbm, *scr):
        idx_v = scr[0]
        pos_v = scr[1:1 + _NSLOT]
        tok_v = scr[1 + _NSLOT:1 + 2 * _NSLOT]
        psem = scr[1 + 2 * _NSLOT:1 + 3 * _NSLOT]
        gsem = scr[1 + 3 * _NSLOT:1 + 4 * _NSLOT]
        osem = scr[1 + 4 * _NSLOT:1 + 5 * _NSLOT]

        wid = lax.axis_index("s") * _NC + lax.axis_index("c")
        s0 = wid * SW

        pltpu.sync_copy(idx_hbm.at[wid], idx_v)

        def pos_load(c, slot):
            pltpu.async_copy(
                pos_hbm.at[pl.ds(s0 + c * _C, _C)], pos_v[slot], psem[slot])

        def pos_wait(slot):
            pltpu.make_async_copy(
                pos_hbm.at[pl.ds(0, _C)], pos_v[slot], psem[slot]).wait()

        def gather(c, slot):
            pltpu.async_copy(
                tok_hbm.at[idx_v.at[pl.ds(c * R, R)]], tok_v[slot],
                gsem[slot])

        def gather_wait(slot):
            pltpu.make_async_copy(
                tok_hbm.at[pl.ds(0, R)], tok_v[slot], gsem[slot]).wait()

        def store(c, b, slot):
            pltpu.async_copy(
                tok_v[slot].at[pl.ds(b * _C, _C)],
                out_hbm.at[pl.ds(b * S + s0 + c * _C, _C)], osem[slot])

        def store_drain(slot):
            for _ in range(Bn):
                pltpu.make_async_copy(
                    tok_v[slot].at[pl.ds(0, _C)],
                    out_hbm.at[pl.ds(0, _C)], osem[slot]).wait()

        def chunk(c, t, drain, pref_gather, pref_pos):
            gather_wait(t)
            pos_wait(t)

            def row(rr, carry):
                r = lax.rem(rr, _C)
                for j in range(VECS):
                    x = pos_v[t][r, pl.ds(j * _L, _L)]
                    plsc.addupdate(tok_v[t].at[rr, pl.ds(j * _L, _L)], x)
                return carry

            lax.fori_loop(0, R, row, 0)
            for b in range(Bn):
                store(c, b, t)
            if pref_gather:
                s2 = (t + 2) % _NSLOT
                if drain:
                    store_drain(s2)
                gather(c + 2, s2)
            if pref_pos:
                pos_load(c + 3, t)

        for c in range(_NSLOT):
            pos_load(c, c)
        gather(0, 0)
        gather(1, 1)
        chunk(0, 0, False, True, True)
        chunk(1, 1, True, True, True)
        chunk(2, 2, True, True, True)

        def body(k, carry):
            for t in range(_NSLOT):
                chunk(3 * k + t, t, True, True, True)
            return carry
        lax.fori_loop(1, NCH // 3 - 1, body, 0)

        chunk(NCH - 5, (NCH - 5) % _NSLOT, True, True, True)
        chunk(NCH - 4, (NCH - 4) % _NSLOT, True, True, True)
        chunk(NCH - 3, (NCH - 3) % _NSLOT, True, True, False)
        chunk(NCH - 2, (NCH - 2) % _NSLOT, False, False, False)
        chunk(NCH - 1, (NCH - 1) % _NSLOT, False, False, False)
        for c in range(NCH - 3, NCH):
            store_drain(c % _NSLOT)

    return embed


def kernel(input_ids, embed_tokens_w, embed_positions_w):
    Bn, S = input_ids.shape
    V, D = embed_tokens_w.shape
    SW = S // _NW
    NCH = SW // _C
    ids_r = jnp.transpose(
        input_ids.reshape(Bn, _NW, NCH, _C), (1, 2, 0, 3)
    ).reshape(_NW, NCH * Bn * _C)
    embed = _build_embed(Bn, S, V, D)
    out = embed(ids_r, embed_tokens_w, embed_positions_w)
    return out.reshape(Bn, S, D)


# speedup vs baseline: 1.5493x; 1.5493x over previous
import functools

import jax
import jax.numpy as jnp
from jax import lax
from jax.experimental import pallas as pl
from jax.experimental.pallas import tpu as pltpu
from jax.experimental.pallas import tpu_sc as plsc

_NC = 2
_NS = 16
_NW = _NC * _NS
_L = 16
_C = 8
_NSLOT = 8


@functools.lru_cache(maxsize=None)
def _build_embed(Bn, S, V, D):
    assert Bn == 4 and S % _NW == 0 and D % _L == 0
    SW = S // _NW
    NCH = SW // _C
    NSUP = NCH // 2
    VECS = D // _L
    BS = Bn * S
    assert NCH % 2 == 0 and NSUP >= 2

    mesh = plsc.VectorSubcoreMesh(core_axis_name="c", subcore_axis_name="s")
    scratch = (
        [pltpu.VMEM((Bn, SW), jnp.int32)]
        + [pltpu.VMEM((_C, D), jnp.float32) for _ in range(2)]
        + [pltpu.VMEM((_C, D), jnp.float32) for _ in range(_NSLOT)]
        + [pltpu.SemaphoreType.DMA for _ in range(2 + 2 * _NSLOT)]
    )

    @functools.partial(
        pl.kernel,
        mesh=mesh,
        out_type=jax.ShapeDtypeStruct((BS, D), jnp.float32),
        scratch_types=scratch,
    )
    def embed(ids_hbm, tok_hbm, pos_hbm, out_hbm, *scr):
        idx_v = scr[0]
        pos_v = scr[1:3]
        tok_v = scr[3:3 + _NSLOT]
        psem = scr[3 + _NSLOT:5 + _NSLOT]
        gsem = scr[5 + _NSLOT:5 + 2 * _NSLOT]
        osem = scr[5 + 2 * _NSLOT:5 + 3 * _NSLOT]

        wid = lax.axis_index("s") * _NC + lax.axis_index("c")
        s0 = wid * SW

        for b in range(Bn):
            pltpu.sync_copy(ids_hbm.at[b, pl.ds(s0, SW)], idx_v.at[b])

        def pos_load(c, cc):
            pltpu.async_copy(
                pos_hbm.at[pl.ds(s0 + c * _C, _C)], pos_v[cc], psem[cc])

        def pos_wait(cc):
            pltpu.make_async_copy(
                pos_hbm.at[pl.ds(0, _C)], pos_v[cc], psem[cc]).wait()

        def gather(i, p):
            cc, b = divmod(p, 4)
            c = 2 * i + cc
            pltpu.async_copy(
                tok_hbm.at[idx_v.at[b, pl.ds(c * _C, _C)]], tok_v[p], gsem[p])

        def gather_wait(p):
            pltpu.make_async_copy(
                tok_hbm.at[pl.ds(0, _C)], tok_v[p], gsem[p]).wait()

        def store(i, p):
            cc, b = divmod(p, 4)
            c = 2 * i + cc
            pltpu.async_copy(
                tok_v[p], out_hbm.at[pl.ds(b * S + s0 + c * _C, _C)], osem[p])

        def store_wait(p):
            pltpu.make_async_copy(
                tok_v[p], out_hbm.at[pl.ds(0, _C)], osem[p]).wait()

        def adds(cc, p):
            def row(r, carry):
                for j in range(VECS):
                    x = pos_v[cc][r, pl.ds(j * _L, _L)]
                    plsc.addupdate(tok_v[p].at[r, pl.ds(j * _L, _L)], x)
                return carry
            lax.fori_loop(0, _C, row, 0)

        def unit(i, p, first_super, last_super):
            cc, b = divmod(p, 4)
            gather_wait(p)
            if p < 4:
                q = p + 4
                if not first_super:
                    store_wait(q)
                gather(i, q)
            else:
                q = p - 4
                if not last_super:
                    store_wait(q)
                    gather(i + 1, q)
            if b == 0:
                pos_wait(cc)
            adds(cc, p)
            store(i, p)
            if b == 3 and not last_super:
                pos_load(2 * (i + 1) + cc, cc)

        pos_load(0, 0)
        pos_load(1, 1)
        for b in range(Bn):
            gather(0, b)
        for p in range(_NSLOT):
            unit(0, p, True, False)
        def body(i, carry):
            for p in range(_NSLOT):
                unit(i, p, False, False)
            return carry
        lax.fori_loop(1, NSUP - 1, body, 0)
        for p in range(_NSLOT):
            unit(NSUP - 1, p, False, True)
        for p in range(_NSLOT):
            store_wait(p)

    return embed


def kernel(input_ids, embed_tokens_w, embed_positions_w):
    Bn, S = input_ids.shape
    V, D = embed_tokens_w.shape
    embed = _build_embed(Bn, S, V, D)
    out = embed(input_ids, embed_tokens_w, embed_positions_w)
    return out.reshape(Bn, S, D)


# speedup vs baseline: 1.5585x; 1.0059x over previous
import functools

import jax
import jax.numpy as jnp
from jax import lax
from jax.experimental import pallas as pl
from jax.experimental.pallas import tpu as pltpu
from jax.experimental.pallas import tpu_sc as plsc

_NC = 2
_NS = 16
_NW = _NC * _NS
_L = 16
_C = 8
_NSLOT = 8


@functools.lru_cache(maxsize=None)
def _build_embed(Bn, S, V, D):
    assert Bn == 4 and S % _NW == 0 and D % _L == 0
    SW = S // _NW
    NCH = SW // _C
    NSUP = NCH // 2
    VECS = D // _L
    BS = Bn * S
    assert NCH % 2 == 0 and NSUP >= 2

    mesh = plsc.VectorSubcoreMesh(core_axis_name="c", subcore_axis_name="s")
    scratch = (
        [pltpu.VMEM((Bn, SW), jnp.int32)]
        + [pltpu.VMEM((_C, D), jnp.float32) for _ in range(2)]
        + [pltpu.VMEM((_C, D), jnp.float32) for _ in range(_NSLOT)]
        + [pltpu.SemaphoreType.DMA for _ in range(2 + 2 * _NSLOT)]
    )

    @functools.partial(
        pl.kernel,
        mesh=mesh,
        out_type=jax.ShapeDtypeStruct((BS, D), jnp.float32),
        scratch_types=scratch,
    )
    def embed(ids_hbm, tok_hbm, pos_hbm, out_hbm, *scr):
        idx_v = scr[0]
        pos_v = scr[1:3]
        tok_v = scr[3:3 + _NSLOT]
        psem = scr[3 + _NSLOT:5 + _NSLOT]
        gsem = scr[5 + _NSLOT:5 + 2 * _NSLOT]
        osem = scr[5 + 2 * _NSLOT:5 + 3 * _NSLOT]

        wid = lax.axis_index("s") * _NC + lax.axis_index("c")
        s0 = wid * SW

        for b in range(Bn):
            pltpu.sync_copy(ids_hbm.at[b, pl.ds(s0, SW)], idx_v.at[b])

        def pos_load(c, cc):
            pltpu.async_copy(
                pos_hbm.at[pl.ds(s0 + c * _C, _C)], pos_v[cc], psem[cc])

        def pos_wait(cc):
            pltpu.make_async_copy(
                pos_hbm.at[pl.ds(0, _C)], pos_v[cc], psem[cc]).wait()

        def gather(i, p):
            cc, b = divmod(p, 4)
            c = 2 * i + cc
            pltpu.async_copy(
                tok_hbm.at[idx_v.at[b, pl.ds(c * _C, _C)]], tok_v[p], gsem[p])

        def gather_wait(p):
            pltpu.make_async_copy(
                tok_hbm.at[pl.ds(0, _C)], tok_v[p], gsem[p]).wait()

        def store(i, p):
            cc, b = divmod(p, 4)
            c = 2 * i + cc
            pltpu.async_copy(
                tok_v[p], out_hbm.at[pl.ds(b * S + s0 + c * _C, _C)], osem[p])

        def store_wait(p):
            pltpu.make_async_copy(
                tok_v[p], out_hbm.at[pl.ds(0, _C)], osem[p]).wait()

        def adds(cc, p):
            @plsc.parallel_loop(0, _C)
            def row(r):
                for j in range(VECS):
                    x = pos_v[cc][r, pl.ds(j * _L, _L)]
                    plsc.addupdate(tok_v[p].at[r, pl.ds(j * _L, _L)], x)

        def unit(i, p, first_super, last_super):
            cc, b = divmod(p, 4)
            gather_wait(p)
            if p < 4:
                q = p + 4
                if not first_super:
                    store_wait(q)
                gather(i, q)
            else:
                q = p - 4
                if not last_super:
                    store_wait(q)
                    gather(i + 1, q)
            if b == 0:
                pos_wait(cc)
            adds(cc, p)
            store(i, p)
            if b == 3 and not last_super:
                pos_load(2 * (i + 1) + cc, cc)

        pos_load(0, 0)
        pos_load(1, 1)
        for b in range(Bn):
            gather(0, b)
        for p in range(_NSLOT):
            unit(0, p, True, False)
        def body(i, carry):
            for p in range(_NSLOT):
                unit(i, p, False, False)
            return carry
        lax.fori_loop(1, NSUP - 1, body, 0)
        for p in range(_NSLOT):
            unit(NSUP - 1, p, False, True)
        for p in range(_NSLOT):
            store_wait(p)

    return embed


def kernel(input_ids, embed_tokens_w, embed_positions_w):
    Bn, S = input_ids.shape
    V, D = embed_tokens_w.shape
    embed = _build_embed(Bn, S, V, D)
    out = embed(input_ids, embed_tokens_w, embed_positions_w)
    return out.reshape(Bn, S, D)
